# baseline (speedup 1.0000x reference)
import functools

import jax
import jax.numpy as jnp
from jax import lax
from jax.experimental import pallas as pl
from jax.experimental.pallas import tpu as pltpu
from jax.experimental.pallas import tpu_sc as plsc

N = 10000
E = 320000
D = 128
MAXD = 8
N_PAD = 10112
N_TBL = 10016
BATCH = 128
NTILES = 32
CHUNK = 10112
E_IN_PAD = NTILES * CHUNK
NBLK_MAX = CHUNK // BATCH
IBUF = 16
NSPLIT = 4
SPB = BATCH // NSPLIT
NBLK_ALLOC = 80
NREG = NTILES * MAXD
HSTG = 256
STG = 2 * HSTG
ROWS_PER_TILE = N_PAD // 16
BM = 1000

_MESH = plsc.VectorSubcoreMesh(core_axis_name="c", subcore_axis_name="s")


@functools.partial(
    pl.kernel,
    out_type=(
        jax.ShapeDtypeStruct((NREG, NBLK_ALLOC, BATCH), jnp.int32),
        jax.ShapeDtypeStruct((NREG, NBLK_ALLOC, BATCH), jnp.int32),
        jax.ShapeDtypeStruct((NTILES, 16), jnp.int32),
    ),
    mesh=_MESH,
    scratch_types=[
        pltpu.VMEM((N_TBL,), jnp.int32),
        pltpu.VMEM((CHUNK,), jnp.int32),
        pltpu.VMEM((CHUNK,), jnp.int32),
    ] + [pltpu.VMEM((STG,), jnp.int32) for _ in range(2 * MAXD)]
      + [pltpu.VMEM((16,), jnp.int32), pltpu.SemaphoreType.DMA],
    compiler_params=pltpu.CompilerParams(needs_layout_passes=False),
)
def _sc_prepass(srcp_hbm, dstp_hbm, deptht_hbm, esrc_o, edst_o, nblk_o,
                depv, bsrc, bdst, *stg_rest):
    ssrc = stg_rest[:MAXD]
    sdst = stg_rest[MAXD:2 * MAXD]
    nbv = stg_rest[2 * MAXD]
    sem = stg_rest[2 * MAXD + 1]
    cid = lax.axis_index("c")
    sid = lax.axis_index("s")
    t = sid * 2 + cid
    gc = pl.multiple_of(t * CHUNK, BATCH)
    cp1 = pltpu.async_copy(srcp_hbm.at[pl.ds(gc, CHUNK)], bsrc, sem)
    cp2 = pltpu.async_copy(dstp_hbm.at[pl.ds(gc, CHUNK)], bdst, sem)
    pltpu.sync_copy(deptht_hbm, depv)
    cp1.wait()
    cp2.wait()
    iota16 = lax.iota(jnp.int32, 16)

    def batch_body(j, carry):
        curs, blks, pend = carry
        curs = list(curs)
        blks = list(blks)
        for sv in range(8):
            sl = pl.ds(j * BATCH + sv * 16, 16)
            dvec = bdst[sl]
            svec = bsrc[sl]
            lvl = plsc.load_gather(depv, [dvec])
            for k in range(MAXD):
                m = lvl == k
                par = jnp.bitwise_and(blks[k], 1)
                woff = par * HSTG + curs[k]
                plsc.store_compressed(ssrc[k].at[pl.ds(woff, 16)], svec,
                                      mask=m)
                plsc.store_compressed(sdst[k].at[pl.ds(woff, 16)], dvec,
                                      mask=m)
                cnt = plsc.all_reduce_population_count(m)[0]
                cur2 = curs[k] + cnt
                full = cur2 >= BATCH

                for p0 in range(2):
                    @pl.when(full & (par == p0))
                    def _flush(k=k, blk=blks[k], pend=pend, p0=p0):
                        rr = t * MAXD + k
                        off = p0 * HSTG
                        noff = (1 - p0) * HSTG

                        @pl.when(pend > 0)
                        def _drain():
                            pltpu.make_async_copy(ssrc[k].at[pl.ds(0, BATCH)],
                                                  esrc_o.at[rr, blk],
                                                  sem).wait()
                            pltpu.make_async_copy(ssrc[k].at[pl.ds(0, BATCH)],
                                                  edst_o.at[rr, blk],
                                                  sem).wait()

                        pltpu.async_copy(ssrc[k].at[pl.ds(off, BATCH)],
                                         esrc_o.at[rr, blk], sem)
                        pltpu.async_copy(sdst[k].at[pl.ds(off, BATCH)],
                                         edst_o.at[rr, blk], sem)
                        ssrc[k][pl.ds(noff, 16)] = ssrc[k][pl.ds(off + BATCH,
                                                                 16)]
                        sdst[k][pl.ds(noff, 16)] = sdst[k][pl.ds(off + BATCH,
                                                                 16)]

                pend = jnp.where(full, jnp.int32(1), pend)
                curs[k] = jnp.where(full, cur2 - BATCH, cur2)
                blks[k] = jnp.where(full, blks[k] + 1, blks[k])
        return tuple(curs), tuple(blks), pend

    zero = jnp.int32(0)
    curs, blks, pend = lax.fori_loop(0, NBLK_MAX, batch_body,
                                     ((zero,) * MAXD, (zero,) * MAXD, zero))

    @pl.when(pend > 0)
    def _drain_last():
        pltpu.make_async_copy(ssrc[0].at[pl.ds(0, BATCH)],
                              esrc_o.at[0, 0], sem).wait()
        pltpu.make_async_copy(ssrc[0].at[pl.ds(0, BATCH)],
                              edst_o.at[0, 0], sem).wait()

    dummy_s = jnp.zeros((16,), jnp.int32)
    dummy_d = jnp.full((16,), N, jnp.int32)
    nbvec = jnp.zeros((16,), jnp.int32)
    for k in range(MAXD):
        cur = curs[k]
        blk = blks[k]
        has = cur > 0

        for p0 in range(2):
            @pl.when(has & (jnp.bitwise_and(blk, 1) == p0))
            def _final(k=k, cur=cur, blk=blk, p0=p0):
                off = p0 * HSTG
                for i in range(8):
                    sl = pl.ds(off + i * 16, 16)
                    mm = (i * 16 + iota16) >= cur
                    ssrc[k][sl] = jnp.where(mm, dummy_s, ssrc[k][sl])
                    sdst[k][sl] = jnp.where(mm, dummy_d, sdst[k][sl])
                rr = t * MAXD + k
                pltpu.sync_copy(ssrc[k].at[pl.ds(off, BATCH)],
                                esrc_o.at[rr, blk])
                pltpu.sync_copy(sdst[k].at[pl.ds(off, BATCH)],
                                edst_o.at[rr, blk])

        nb_k = blk + jnp.where(has, 1, 0)
        nbvec = jnp.where(iota16 == k, nb_k, nbvec)
    nbv[...] = nbvec
    pltpu.sync_copy(nbv, nblk_o.at[t])


def _make_sc_level(k: int):
    @functools.partial(
        pl.kernel,
        out_type=jax.ShapeDtypeStruct((2, N_PAD, D), jnp.float32),
        mesh=_MESH,
        scratch_types=[
            pltpu.VMEM((2, IBUF, BATCH), jnp.int32),
            pltpu.VMEM((2, IBUF, BATCH), jnp.int32),
            pltpu.VMEM((2, BATCH, D), jnp.float32),
            pltpu.VMEM((16,), jnp.int32),
            pltpu.VMEM_SHARED((N_PAD, D), jnp.float32),
            pltpu.SemaphoreType.DMA,
            pltpu.SemaphoreType.DMA,
        ],
    )
    def sc_level(h_hbm, esrc_hbm, edst_hbm, nblk_hbm, zeros_hbm, out_hbm,
                 idx_s2, idx_d2, rows2, nbv, agg_sh, semi, semg):
        cid = lax.axis_index("c")
        sid = lax.axis_index("s")
        t = sid * 2 + cid
        rr = t * MAXD + k

        pltpu.sync_copy(zeros_hbm.at[pl.ds(sid * ROWS_PER_TILE, ROWS_PER_TILE)],
                        agg_sh.at[pl.ds(sid * ROWS_PER_TILE, ROWS_PER_TILE)])
        pltpu.sync_copy(nblk_hbm.at[t], nbv)
        nb = nbv[...][k]
        ngrp = (nb + IBUF - 1) // IBUF
        plsc.subcore_barrier()

        @pl.when(ngrp > 0)
        def _prime_idx():
            pltpu.async_copy(esrc_hbm.at[rr, pl.ds(0, IBUF)], idx_s2.at[0],
                             semi)
            pltpu.async_copy(edst_hbm.at[rr, pl.ds(0, IBUF)], idx_d2.at[0],
                             semi)

        def grp_body(gidx, carry):
            gbuf = lax.rem(gidx, 2)
            base = gidx * IBUF
            pltpu.make_async_copy(esrc_hbm.at[rr, pl.ds(0, IBUF)],
                                  idx_s2.at[gbuf], semi).wait()
            pltpu.make_async_copy(edst_hbm.at[rr, pl.ds(0, IBUF)],
                                  idx_d2.at[gbuf], semi).wait()

            @pl.when(base + IBUF < nb)
            def _prefetch_idx():
                off = pl.multiple_of((gidx + 1) * IBUF, IBUF)
                pltpu.async_copy(esrc_hbm.at[rr, pl.ds(off, IBUF)],
                                 idx_s2.at[1 - gbuf], semi)
                pltpu.async_copy(edst_hbm.at[rr, pl.ds(off, IBUF)],
                                 idx_d2.at[1 - gbuf], semi)

            @pl.when(base < nb)
            def _prime_gather():
                for sp in range(NSPLIT):
                    pltpu.async_copy(
                        h_hbm.at[idx_s2.at[gbuf, 0, pl.ds(sp * SPB, SPB)]],
                        rows2.at[0, pl.ds(sp * SPB, SPB)], semg)

            for i in range(IBUF):
                j = base + i

                @pl.when((j + 1 < nb) & (i + 1 < IBUF))
                def _prefetch_gather(i=i):
                    for sp in range(NSPLIT):
                        pltpu.async_copy(
                            h_hbm.at[idx_s2.at[gbuf, i + 1,
                                               pl.ds(sp * SPB, SPB)]],
                            rows2.at[(i + 1) % 2, pl.ds(sp * SPB, SPB)], semg)

                @pl.when(j < nb)
                def _consume(i=i):
                    pltpu.make_async_copy(h_hbm.at[idx_s2.at[gbuf, i]],
                                          rows2.at[i % 2], semg).wait()
                    pltpu.sync_copy(rows2.at[i % 2],
                                    agg_sh.at[idx_d2.at[gbuf, i]], add=True)
            return carry

        lax.fori_loop(0, ngrp, grp_body, 0)
        plsc.subcore_barrier()
        pltpu.sync_copy(agg_sh.at[pl.ds(sid * ROWS_PER_TILE, ROWS_PER_TILE)],
                        out_hbm.at[cid, pl.ds(sid * ROWS_PER_TILE, ROWS_PER_TILE)])

    return sc_level


def _mm_bias_body(x_ref, w_ref, b_ref, o_ref):
    o_ref[...] = (jnp.dot(x_ref[...], w_ref[...],
                          preferred_element_type=jnp.float32) + b_ref[...])


def _tc_matmul_bias(x, W, b):
    m = x.shape[0]
    return pl.pallas_call(
        _mm_bias_body,
        grid=(m // BM,),
        in_specs=[
            pl.BlockSpec((BM, D), lambda i: (i, 0)),
            pl.BlockSpec((D, D), lambda i: (0, 0)),
            pl.BlockSpec((1, D), lambda i: (0, 0)),
        ],
        out_specs=pl.BlockSpec((BM, D), lambda i: (i, 0)),
        out_shape=jax.ShapeDtypeStruct((m, D), jnp.float32),
    )(x, W, b.reshape(1, D))


def _make_lvl_body(k: int):
    def body(h_ref, a0_ref, a1_ref, w1_ref, w2_ref, b_ref, dep_ref, o_ref):
        agg = a0_ref[0] + a1_ref[0]
        newh = jnp.maximum(
            jnp.dot(h_ref[...], w1_ref[...], preferred_element_type=jnp.float32)
            + jnp.dot(agg, w2_ref[...], preferred_element_type=jnp.float32)
            + b_ref[...], 0.0)
        o_ref[...] = jnp.where(dep_ref[...] == k, newh, h_ref[...])
    return body


def _tc_level_update(k, h, agg2, W1, W2, b_lin, depb):
    return pl.pallas_call(
        _make_lvl_body(k),
        grid=(N // BM,),
        in_specs=[
            pl.BlockSpec((BM, D), lambda i: (i, 0)),
            pl.BlockSpec((1, BM, D), lambda i: (0, i, 0)),
            pl.BlockSpec((1, BM, D), lambda i: (1, i, 0)),
            pl.BlockSpec((D, D), lambda i: (0, 0)),
            pl.BlockSpec((D, D), lambda i: (0, 0)),
            pl.BlockSpec((1, D), lambda i: (0, 0)),
            pl.BlockSpec((BM, D), lambda i: (i, 0)),
        ],
        out_specs=pl.BlockSpec((BM, D), lambda i: (i, 0)),
        out_shape=jax.ShapeDtypeStruct((N, D), jnp.float32),
    )(h, agg2, agg2, W1, W2, b_lin.reshape(1, D), depb)


def kernel(x, edge_index, depth, W_in, b_in, W_lin, b_lin, W_out, b_out):
    src = edge_index[0]
    dst = edge_index[1]
    srcp = jnp.concatenate([src, jnp.zeros((E_IN_PAD - E,), jnp.int32)])
    dstp = jnp.concatenate([dst, jnp.full((E_IN_PAD - E,), N, jnp.int32)])
    deptht = jnp.concatenate([depth, jnp.full((N_TBL - N,), MAXD, jnp.int32)])

    esrc, edst, nblk = _sc_prepass(srcp, dstp, deptht)

    zeros_hbm = jnp.zeros((N_PAD, D), jnp.float32)
    depb = jnp.broadcast_to(depth[:, None], (N, D))

    h = _tc_matmul_bias(x, W_in, b_in)
    W1 = W_lin[:D]
    W2 = W_lin[D:]
    for k in range(MAXD):
        agg2 = _make_sc_level(k)(h, esrc, edst, nblk, zeros_hbm)
        h = _tc_level_update(k, h, agg2, W1, W2, b_lin, depb)
    z = _tc_matmul_bias(h, W_out, b_out)
    return z


# speedup vs baseline: 1.2104x; 1.2104x over previous
import functools

import jax
import jax.numpy as jnp
from jax import lax
from jax.experimental import pallas as pl
from jax.experimental.pallas import tpu as pltpu
from jax.experimental.pallas import tpu_sc as plsc

N = 10000
E = 320000
D = 128
MAXD = 8
N_PAD = 10112
N_TBL = 10016
BATCH = 128
NTILES = 32
CHUNK = 10112
E_IN_PAD = NTILES * CHUNK
NBLK_MAX = CHUNK // BATCH
IBUF = 16
NSPLIT = 4
SPB = BATCH // NSPLIT
NBLK_ALLOC = 80
NREG = NTILES * MAXD
STG = 160
ROWS_PER_TILE = N_PAD // 16
BM = 1000

_MESH = plsc.VectorSubcoreMesh(core_axis_name="c", subcore_axis_name="s")


@functools.partial(
    pl.kernel,
    out_type=(
        jax.ShapeDtypeStruct((NREG, NBLK_ALLOC, BATCH), jnp.int32),
        jax.ShapeDtypeStruct((NREG, NBLK_ALLOC, BATCH), jnp.int32),
        jax.ShapeDtypeStruct((NTILES, 16), jnp.int32),
    ),
    mesh=_MESH,
    scratch_types=[
        pltpu.VMEM((N_TBL,), jnp.int32),
        pltpu.VMEM((CHUNK,), jnp.int32),
        pltpu.VMEM((CHUNK,), jnp.int32),
    ] + [pltpu.VMEM((STG,), jnp.int32) for _ in range(2 * MAXD)]
      + [pltpu.VMEM((16,), jnp.int32), pltpu.SemaphoreType.DMA],
    compiler_params=pltpu.CompilerParams(needs_layout_passes=False),
)
def _sc_prepass(srcp_hbm, dstp_hbm, deptht_hbm, esrc_o, edst_o, nblk_o,
                depv, bsrc, bdst, *stg_rest):
    ssrc = stg_rest[:MAXD]
    sdst = stg_rest[MAXD:2 * MAXD]
    nbv = stg_rest[2 * MAXD]
    sem = stg_rest[2 * MAXD + 1]
    cid = lax.axis_index("c")
    sid = lax.axis_index("s")
    t = sid * 2 + cid
    gc = pl.multiple_of(t * CHUNK, BATCH)
    cp1 = pltpu.async_copy(srcp_hbm.at[pl.ds(gc, CHUNK)], bsrc, sem)
    cp2 = pltpu.async_copy(dstp_hbm.at[pl.ds(gc, CHUNK)], bdst, sem)
    pltpu.sync_copy(deptht_hbm, depv)
    cp1.wait()
    cp2.wait()
    iota16 = lax.iota(jnp.int32, 16)

    def batch_body(j, carry):
        curs, blks = carry
        curs = list(curs)
        blks = list(blks)
        for sv in range(8):
            sl = pl.ds(j * BATCH + sv * 16, 16)
            dvec = bdst[sl]
            svec = bsrc[sl]
            lvl = plsc.load_gather(depv, [dvec])
            for k in range(MAXD):
                m = lvl == k
                plsc.store_compressed(ssrc[k].at[pl.ds(curs[k], 16)], svec,
                                      mask=m)
                plsc.store_compressed(sdst[k].at[pl.ds(curs[k], 16)], dvec,
                                      mask=m)
                cnt = plsc.all_reduce_population_count(m)[0]
                cur2 = curs[k] + cnt
                full = cur2 >= BATCH

                @pl.when(full)
                def _flush(k=k, blk=blks[k]):
                    rr = t * MAXD + k
                    pltpu.sync_copy(ssrc[k].at[pl.ds(0, BATCH)],
                                    esrc_o.at[rr, blk])
                    pltpu.sync_copy(sdst[k].at[pl.ds(0, BATCH)],
                                    edst_o.at[rr, blk])
                    ssrc[k][pl.ds(0, 16)] = ssrc[k][pl.ds(BATCH, 16)]
                    sdst[k][pl.ds(0, 16)] = sdst[k][pl.ds(BATCH, 16)]

                curs[k] = jnp.where(full, cur2 - BATCH, cur2)
                blks[k] = jnp.where(full, blks[k] + 1, blks[k])
        return tuple(curs), tuple(blks)

    zero = jnp.int32(0)
    curs, blks = lax.fori_loop(0, NBLK_MAX, batch_body,
                               ((zero,) * MAXD, (zero,) * MAXD))

    dummy_s = jnp.zeros((16,), jnp.int32)
    dummy_d = jnp.full((16,), N, jnp.int32)
    nbvec = jnp.zeros((16,), jnp.int32)
    for k in range(MAXD):
        cur = curs[k]
        blk = blks[k]
        has = cur > 0

        @pl.when(has)
        def _final(k=k, cur=cur, blk=blk):
            for i in range(8):
                sl = pl.ds(i * 16, 16)
                mm = (i * 16 + iota16) >= cur
                ssrc[k][sl] = jnp.where(mm, dummy_s, ssrc[k][sl])
                sdst[k][sl] = jnp.where(mm, dummy_d, sdst[k][sl])
            rr = t * MAXD + k
            pltpu.sync_copy(ssrc[k].at[pl.ds(0, BATCH)], esrc_o.at[rr, blk])
            pltpu.sync_copy(sdst[k].at[pl.ds(0, BATCH)], edst_o.at[rr, blk])

        nb_k = blk + jnp.where(has, 1, 0)
        nbvec = jnp.where(iota16 == k, nb_k, nbvec)
    nbv[...] = nbvec
    pltpu.sync_copy(nbv, nblk_o.at[t])


def _make_sc_level(k: int):
    @functools.partial(
        pl.kernel,
        out_type=jax.ShapeDtypeStruct((2, N_PAD, D), jnp.float32),
        mesh=_MESH,
        scratch_types=[
            pltpu.VMEM((2, IBUF, BATCH), jnp.int32),
            pltpu.VMEM((2, IBUF, BATCH), jnp.int32),
            pltpu.VMEM((2, BATCH, D), jnp.float32),
            pltpu.VMEM((16,), jnp.int32),
            pltpu.VMEM_SHARED((N_PAD, D), jnp.float32),
            pltpu.SemaphoreType.DMA,
            pltpu.SemaphoreType.DMA,
        ],
    )
    def sc_level(h_hbm, esrc_hbm, edst_hbm, nblk_hbm, zeros_hbm, out_hbm,
                 idx_s2, idx_d2, rows2, nbv, agg_sh, semi, semg):
        cid = lax.axis_index("c")
        sid = lax.axis_index("s")
        t = sid * 2 + cid
        rr = t * MAXD + k

        pltpu.sync_copy(zeros_hbm.at[pl.ds(sid * ROWS_PER_TILE, ROWS_PER_TILE)],
                        agg_sh.at[pl.ds(sid * ROWS_PER_TILE, ROWS_PER_TILE)])
        pltpu.sync_copy(nblk_hbm.at[t], nbv)
        nb = nbv[...][k]
        ngrp = (nb + IBUF - 1) // IBUF
        plsc.subcore_barrier()

        @pl.when(ngrp > 0)
        def _prime_idx():
            pltpu.async_copy(esrc_hbm.at[rr, pl.ds(0, IBUF)], idx_s2.at[0],
                             semi)
            pltpu.async_copy(edst_hbm.at[rr, pl.ds(0, IBUF)], idx_d2.at[0],
                             semi)

        def grp_body(gidx, carry):
            gbuf = lax.rem(gidx, 2)
            base = gidx * IBUF
            pltpu.make_async_copy(esrc_hbm.at[rr, pl.ds(0, IBUF)],
                                  idx_s2.at[gbuf], semi).wait()
            pltpu.make_async_copy(edst_hbm.at[rr, pl.ds(0, IBUF)],
                                  idx_d2.at[gbuf], semi).wait()

            @pl.when(base + IBUF < nb)
            def _prefetch_idx():
                off = pl.multiple_of((gidx + 1) * IBUF, IBUF)
                pltpu.async_copy(esrc_hbm.at[rr, pl.ds(off, IBUF)],
                                 idx_s2.at[1 - gbuf], semi)
                pltpu.async_copy(edst_hbm.at[rr, pl.ds(off, IBUF)],
                                 idx_d2.at[1 - gbuf], semi)

            @pl.when(base < nb)
            def _prime_gather():
                for sp in range(NSPLIT):
                    pltpu.async_copy(
                        h_hbm.at[idx_s2.at[gbuf, 0, pl.ds(sp * SPB, SPB)]],
                        rows2.at[0, pl.ds(sp * SPB, SPB)], semg)

            for i in range(IBUF):
                j = base + i

                @pl.when((j + 1 < nb) & (i + 1 < IBUF))
                def _prefetch_gather(i=i):
                    for sp in range(NSPLIT):
                        pltpu.async_copy(
                            h_hbm.at[idx_s2.at[gbuf, i + 1,
                                               pl.ds(sp * SPB, SPB)]],
                            rows2.at[(i + 1) % 2, pl.ds(sp * SPB, SPB)], semg)

                @pl.when(j < nb)
                def _consume(i=i):
                    pltpu.make_async_copy(h_hbm.at[idx_s2.at[gbuf, i]],
                                          rows2.at[i % 2], semg).wait()
                    pltpu.sync_copy(rows2.at[i % 2],
                                    agg_sh.at[idx_d2.at[gbuf, i]], add=True)
            return carry

        lax.fori_loop(0, ngrp, grp_body, 0)
        plsc.subcore_barrier()
        pltpu.sync_copy(agg_sh.at[pl.ds(sid * ROWS_PER_TILE, ROWS_PER_TILE)],
                        out_hbm.at[cid, pl.ds(sid * ROWS_PER_TILE, ROWS_PER_TILE)])

    return sc_level


def _mm_bias_body(x_ref, w_ref, b_ref, o_ref):
    o_ref[...] = (jnp.dot(x_ref[...], w_ref[...],
                          preferred_element_type=jnp.float32) + b_ref[...])


def _tc_matmul_bias(x, W, b):
    m = x.shape[0]
    return pl.pallas_call(
        _mm_bias_body,
        grid=(m // BM,),
        in_specs=[
            pl.BlockSpec((BM, D), lambda i: (i, 0)),
            pl.BlockSpec((D, D), lambda i: (0, 0)),
            pl.BlockSpec((1, D), lambda i: (0, 0)),
        ],
        out_specs=pl.BlockSpec((BM, D), lambda i: (i, 0)),
        out_shape=jax.ShapeDtypeStruct((m, D), jnp.float32),
    )(x, W, b.reshape(1, D))


def _make_lvl_body(k: int):
    def body(h_ref, a0_ref, a1_ref, w1_ref, w2_ref, b_ref, dep_ref, o_ref):
        agg = a0_ref[0] + a1_ref[0]
        newh = jnp.maximum(
            jnp.dot(h_ref[...], w1_ref[...], preferred_element_type=jnp.float32)
            + jnp.dot(agg, w2_ref[...], preferred_element_type=jnp.float32)
            + b_ref[...], 0.0)
        o_ref[...] = jnp.where(dep_ref[...] == k, newh, h_ref[...])
    return body


def _tc_level_update(k, h, agg2, W1, W2, b_lin, depb):
    return pl.pallas_call(
        _make_lvl_body(k),
        grid=(N // BM,),
        in_specs=[
            pl.BlockSpec((BM, D), lambda i: (i, 0)),
            pl.BlockSpec((1, BM, D), lambda i: (0, i, 0)),
            pl.BlockSpec((1, BM, D), lambda i: (1, i, 0)),
            pl.BlockSpec((D, D), lambda i: (0, 0)),
            pl.BlockSpec((D, D), lambda i: (0, 0)),
            pl.BlockSpec((1, D), lambda i: (0, 0)),
            pl.BlockSpec((BM, D), lambda i: (i, 0)),
        ],
        out_specs=pl.BlockSpec((BM, D), lambda i: (i, 0)),
        out_shape=jax.ShapeDtypeStruct((N, D), jnp.float32),
    )(h, agg2, agg2, W1, W2, b_lin.reshape(1, D), depb)


def kernel(x, edge_index, depth, W_in, b_in, W_lin, b_lin, W_out, b_out):
    src = edge_index[0]
    dst = edge_index[1]
    srcp = jnp.concatenate([src, jnp.zeros((E_IN_PAD - E,), jnp.int32)])
    dstp = jnp.concatenate([dst, jnp.full((E_IN_PAD - E,), N, jnp.int32)])
    deptht = jnp.concatenate([depth, jnp.full((N_TBL - N,), MAXD, jnp.int32)])

    esrc, edst, nblk = _sc_prepass(srcp, dstp, deptht)

    zeros_hbm = jnp.zeros((N_PAD, D), jnp.float32)
    depb = jnp.broadcast_to(depth[:, None], (N, D))

    h = _tc_matmul_bias(x, W_in, b_in)
    W1 = W_lin[:D]
    W2 = W_lin[D:]
    for k in range(MAXD):
        agg2 = _make_sc_level(k)(h, esrc, edst, nblk, zeros_hbm)
        h = _tc_level_update(k, h, agg2, W1, W2, b_lin, depb)
    z = _tc_matmul_bias(h, W_out, b_out)
    return z
